# Initial kernel scaffold; baseline (speedup 1.0000x reference)
import functools

import jax
import jax.numpy as jnp
from jax import lax
from jax.experimental import pallas as pl
from jax.experimental.pallas import tpu as pltpu
from jax.experimental.pallas import tpu_sc as plsc

_D = 64
_NEG = 5
_K = 1 + _NEG
_B = 16384
_L = 16
_NC = 2
_NS = 16
_NW = _NC * _NS
_BW = _B // _NW
_CHUNK = 128
_NCHUNK = _BW // _CHUNK

_mesh = plsc.VectorSubcoreMesh(
    core_axis_name="c", subcore_axis_name="s",
    num_cores=_NC, num_subcores=_NS)


@functools.partial(
    pl.kernel,
    out_type=jax.ShapeDtypeStruct((_B, _K * _L), jnp.float32),
    mesh=_mesh,
    scratch_types=[
        pltpu.VMEM((_BW,), jnp.int32),
        pltpu.VMEM((_K, _BW), jnp.int32),
        pltpu.VMEM((_NEG, _BW), jnp.float32),
        pltpu.VMEM((_CHUNK, _D), jnp.float32),
        pltpu.VMEM((_K, _CHUNK, _D), jnp.float32),
        pltpu.VMEM((_CHUNK, _K * _L), jnp.float32),
        pltpu.SemaphoreType.DMA,
    ],
)
def _sc_partials(idx_w, idx_cn, mask, emb0, emb1, out,
                 idx_w_v, idx_cn_v, mask_v, w_v, cn_v, t_v, sem):
    wid = lax.axis_index("s") * _NC + lax.axis_index("c")
    base = wid * _BW
    pltpu.sync_copy(idx_w.at[pl.ds(base, _BW)], idx_w_v)
    pltpu.sync_copy(idx_cn.at[wid], idx_cn_v)
    pltpu.sync_copy(mask.at[wid], mask_v)

    def chunk(ci, carry):
        off = pl.multiple_of(ci * _CHUNK, _CHUNK)
        cps = [pltpu.async_copy(
            emb0.at[idx_w_v.at[pl.ds(off, _CHUNK)]], w_v, sem)]
        for j in range(_K):
            cps.append(pltpu.async_copy(
                emb1.at[idx_cn_v.at[j, pl.ds(off, _CHUNK)]], cn_v.at[j], sem))
        for cp in cps:
            cp.wait()

        def elem(i, c2):
            w0 = w_v[i, pl.ds(0, _L)]
            w1 = w_v[i, pl.ds(_L, _L)]
            w2 = w_v[i, pl.ds(2 * _L, _L)]
            w3 = w_v[i, pl.ds(3 * _L, _L)]
            for j in range(_K):
                t = (w0 * cn_v[j, i, pl.ds(0, _L)]
                     + w1 * cn_v[j, i, pl.ds(_L, _L)]
                     + w2 * cn_v[j, i, pl.ds(2 * _L, _L)]
                     + w3 * cn_v[j, i, pl.ds(3 * _L, _L)])
                if j > 0:
                    t = t * mask_v[j - 1, off + i]
                t_v[i, pl.ds(j * _L, _L)] = t
            return c2

        lax.fori_loop(0, _CHUNK, elem, 0)
        pltpu.sync_copy(t_v, out.at[pl.ds(base + off, _CHUNK)])
        return carry

    lax.fori_loop(0, _NCHUNK, chunk, 0)


def _tc_loss(t_ref, pos_ref, neg_ref):
    x = t_ref[...]
    pos_ip = jnp.sum(x[:, 0:_L], axis=1, keepdims=True)
    pos_ref[0, 0] = jnp.sum(-jax.nn.log_sigmoid(pos_ip))
    acc = jnp.float32(0.0)
    for j in range(1, _K):
        ip = jnp.sum(x[:, j * _L:(j + 1) * _L], axis=1, keepdims=True)
        acc = acc + jnp.sum(-jax.nn.log_sigmoid(-ip))
    neg_ref[0, 0] = acc


_tc_call = pl.pallas_call(
    _tc_loss,
    out_shape=(jax.ShapeDtypeStruct((1, 1), jnp.float32),
               jax.ShapeDtypeStruct((1, 1), jnp.float32)),
    out_specs=(pl.BlockSpec(memory_space=pltpu.SMEM),
               pl.BlockSpec(memory_space=pltpu.SMEM)),
)


def kernel(data, emb0, emb1):
    idx_w = data[:, 0].astype(jnp.int32)
    idx_cn = (data[:, 1:1 + _K].astype(jnp.int32)
              .T.reshape(_K, _NW, _BW).transpose(1, 0, 2))
    mask = (data[:, 1 + _K:].astype(jnp.float32)
            .T.reshape(_NEG, _NW, _BW).transpose(1, 0, 2))
    t = _sc_partials(idx_w, idx_cn, mask, emb0, emb1)
    pos, neg = _tc_call(t)
    return (pos[0, 0], neg[0, 0])


# speedup vs baseline: 1.6240x; 1.6240x over previous
import functools

import jax
import jax.numpy as jnp
from jax import lax
from jax.experimental import pallas as pl
from jax.experimental.pallas import tpu as pltpu
from jax.experimental.pallas import tpu_sc as plsc

_D = 64
_NEG = 5
_K = 1 + _NEG
_B = 16384
_L = 16
_NC = 2
_NS = 16
_NW = _NC * _NS
_BW = _B // _NW
_CHUNK = 128
_NCHUNK = _BW // _CHUNK

_mesh = plsc.VectorSubcoreMesh(
    core_axis_name="c", subcore_axis_name="s",
    num_cores=_NC, num_subcores=_NS)


@functools.partial(
    pl.kernel,
    out_type=jax.ShapeDtypeStruct((_B, _K * _L), jnp.float32),
    mesh=_mesh,
    compiler_params=pltpu.CompilerParams(use_tc_tiling_on_sc=False),
    scratch_types=[
        pltpu.VMEM((_BW,), jnp.int32),
        pltpu.VMEM((_K, _BW), jnp.int32),
        pltpu.VMEM((_CHUNK, _D), jnp.float32),
        pltpu.VMEM((_K, _CHUNK, _D), jnp.float32),
        pltpu.VMEM((_CHUNK, _K * _L), jnp.float32),
        pltpu.SemaphoreType.DMA,
    ],
)
def _sc_partials(idx_w, idx_cn, emb0, emb1, out,
                 idx_w_v, idx_cn_v, w_v, cn_v, t_v, sem):
    wid = lax.axis_index("s") * _NC + lax.axis_index("c")
    base = wid * _BW
    pltpu.sync_copy(idx_w.at[pl.ds(base, _BW)], idx_w_v)
    pltpu.sync_copy(idx_cn.at[wid], idx_cn_v)

    def chunk(ci, carry):
        off = pl.multiple_of(ci * _CHUNK, _CHUNK)
        cps = [pltpu.async_copy(
            emb0.at[idx_w_v.at[pl.ds(off, _CHUNK)]], w_v, sem)]
        for j in range(_K):
            cps.append(pltpu.async_copy(
                emb1.at[idx_cn_v.at[j, pl.ds(off, _CHUNK)]], cn_v.at[j], sem))
        for cp in cps:
            cp.wait()

        def elem(i, c2):
            w0 = w_v[i, pl.ds(0, _L)]
            w1 = w_v[i, pl.ds(_L, _L)]
            w2 = w_v[i, pl.ds(2 * _L, _L)]
            w3 = w_v[i, pl.ds(3 * _L, _L)]
            for j in range(_K):
                t = (w0 * cn_v[j, i, pl.ds(0, _L)]
                     + w1 * cn_v[j, i, pl.ds(_L, _L)]
                     + w2 * cn_v[j, i, pl.ds(2 * _L, _L)]
                     + w3 * cn_v[j, i, pl.ds(3 * _L, _L)])
                t_v[i, pl.ds(j * _L, _L)] = t
            return c2

        lax.fori_loop(0, _CHUNK, elem, 0)
        pltpu.sync_copy(t_v, out.at[pl.ds(base + off, _CHUNK)])
        return carry

    lax.fori_loop(0, _NCHUNK, chunk, 0)


_TC_BLK = 2048
_TC_GRID = _B // _TC_BLK


def _tc_loss(t_ref, mask_ref, pos_ref, neg_ref):
    step = pl.program_id(0)
    x = t_ref[...]
    m = mask_ref[...]
    pos_ip = jnp.sum(x[:, 0:_L], axis=1, keepdims=True)
    pos_part = jnp.sum(-jax.nn.log_sigmoid(pos_ip))
    neg_part = jnp.float32(0.0)
    for j in range(1, _K):
        ip = jnp.sum(x[:, j * _L:(j + 1) * _L], axis=1, keepdims=True)
        ip = ip * m[:, j - 1:j]
        neg_part = neg_part + jnp.sum(-jax.nn.log_sigmoid(-ip))

    @pl.when(step == 0)
    def _():
        pos_ref[0, 0] = jnp.float32(0.0)
        neg_ref[0, 0] = jnp.float32(0.0)

    pos_ref[0, 0] += pos_part
    neg_ref[0, 0] += neg_part


_tc_call = pl.pallas_call(
    _tc_loss,
    grid=(_TC_GRID,),
    in_specs=[
        pl.BlockSpec((_TC_BLK, _K * _L), lambda i: (i, 0)),
        pl.BlockSpec((_TC_BLK, _NEG), lambda i: (i, 0)),
    ],
    out_specs=(
        pl.BlockSpec((1, 1), lambda i: (0, 0), memory_space=pltpu.SMEM),
        pl.BlockSpec((1, 1), lambda i: (0, 0), memory_space=pltpu.SMEM),
    ),
    out_shape=(jax.ShapeDtypeStruct((1, 1), jnp.float32),
               jax.ShapeDtypeStruct((1, 1), jnp.float32)),
)


def kernel(data, emb0, emb1):
    idx_w = data[:, 0].astype(jnp.int32)
    idx_cn = (data[:, 1:1 + _K].astype(jnp.int32)
              .T.reshape(_K, _NW, _BW).transpose(1, 0, 2))
    mask = data[:, 1 + _K:].astype(jnp.float32)
    t = _sc_partials(idx_w, idx_cn, emb0, emb1)
    pos, neg = _tc_call(t, mask)
    return (pos[0, 0], neg[0, 0])
